# baseline (speedup 1.0000x reference)
import jax
import jax.numpy as jnp
from jax.experimental import pallas as pl

NUM_FEATS = 128
CB = 32


def _pos_enc_kernel(colt_ref, rowt_ref, out_ref, *, h, w, half_blocks):
    k = pl.program_id(0)

    @pl.when(k < half_blocks)
    def _():
        slab = jnp.broadcast_to(colt_ref[...][:, None, :], (CB, 8, w))
        nfull = h // 8
        for r in range(nfull):
            out_ref[:, r * 8:(r + 1) * 8, :] = slab
        if h % 8:
            out_ref[:, nfull * 8:h, :] = slab[:, : h % 8, :]

    @pl.when(k >= half_blocks)
    def _():
        rvals = rowt_ref[...]
        nfull = h // 8
        for r in range(nfull):
            out_ref[:, r * 8:(r + 1) * 8, :] = jnp.broadcast_to(
                rvals[:, r * 8:(r + 1) * 8, None], (CB, 8, w))
        if h % 8:
            out_ref[:, nfull * 8:h, :] = jnp.broadcast_to(
                rvals[:, nfull * 8:h, None], (CB, h % 8, w))


def kernel(bev_mask, row_embed, col_embed):
    b = bev_mask.shape[0]
    h, w = bev_mask.shape[-2], bev_mask.shape[-1]
    nf = row_embed.shape[1]
    half_blocks = nf // CB
    nblocks = 2 * half_blocks

    col_t = col_embed[:w].T
    row_t = row_embed[:h].T

    import functools
    body = functools.partial(_pos_enc_kernel, h=h, w=w, half_blocks=half_blocks)

    out = pl.pallas_call(
        body,
        grid=(nblocks,),
        in_specs=[
            pl.BlockSpec((CB, w), lambda k: (jnp.minimum(k, half_blocks - 1), 0)),
            pl.BlockSpec((CB, h), lambda k: (jnp.maximum(k - half_blocks, 0), 0)),
        ],
        out_specs=pl.BlockSpec((CB, h, w), lambda k: (k, 0, 0)),
        out_shape=jax.ShapeDtypeStruct((2 * nf, h, w), jnp.float32),
    )(col_t, row_t)
    return jnp.broadcast_to(out[None], (b, 2 * nf, h, w))


# speedup vs baseline: 3.6278x; 3.6278x over previous
import jax
import jax.numpy as jnp
from jax.experimental import pallas as pl

IB = 8


def _pos_enc_kernel(row_ref, col_ref, out_ref, *, w, nf):
    ce = col_ref[...]
    out_ref[:, :, 0:nf] = jnp.broadcast_to(ce[None], (IB, w, nf))
    rv = row_ref[...]
    out_ref[:, :, nf:2 * nf] = jnp.broadcast_to(rv[:, None, :], (IB, w, nf))


def kernel(bev_mask, row_embed, col_embed):
    b = bev_mask.shape[0]
    h, w = bev_mask.shape[-2], bev_mask.shape[-1]
    nf = row_embed.shape[1]

    import functools
    body = functools.partial(_pos_enc_kernel, w=w, nf=nf)

    grid = (h + IB - 1) // IB
    pos = pl.pallas_call(
        body,
        grid=(grid,),
        in_specs=[
            pl.BlockSpec((IB, nf), lambda i: (i, 0)),
            pl.BlockSpec((w, nf), lambda i: (0, 0)),
        ],
        out_specs=pl.BlockSpec((IB, w, 2 * nf), lambda i: (i, 0, 0)),
        out_shape=jax.ShapeDtypeStruct((h, w, 2 * nf), jnp.float32),
    )(row_embed[:h], col_embed[:w])
    out = jnp.transpose(pos, (2, 0, 1))[None]
    return jnp.broadcast_to(out, (b, 2 * nf, h, w))


# speedup vs baseline: 5.0799x; 1.4003x over previous
import jax
import jax.numpy as jnp
from jax.experimental import pallas as pl

IB = 24


def _pos_enc_kernel(row_ref, col_ref, out_ref, *, w, nf):
    ce = col_ref[...]
    out_ref[:, :, 0:nf] = jnp.broadcast_to(ce[None], (IB, w, nf))
    rv = row_ref[...]
    out_ref[:, :, nf:2 * nf] = jnp.broadcast_to(rv[:, None, :], (IB, w, nf))


def kernel(bev_mask, row_embed, col_embed):
    b = bev_mask.shape[0]
    h, w = bev_mask.shape[-2], bev_mask.shape[-1]
    nf = row_embed.shape[1]

    import functools
    body = functools.partial(_pos_enc_kernel, w=w, nf=nf)

    grid = (h + IB - 1) // IB
    pos = pl.pallas_call(
        body,
        grid=(grid,),
        in_specs=[
            pl.BlockSpec((IB, nf), lambda i: (i, 0)),
            pl.BlockSpec((w, nf), lambda i: (0, 0)),
        ],
        out_specs=pl.BlockSpec((IB, w, 2 * nf), lambda i: (i, 0, 0)),
        out_shape=jax.ShapeDtypeStruct((h, w, 2 * nf), jnp.float32),
    )(row_embed[:h], col_embed[:w])
    out = jnp.transpose(pos, (2, 0, 1))[None]
    return jnp.broadcast_to(out, (b, 2 * nf, h, w))
